# Initial kernel scaffold; baseline (speedup 1.0000x reference)
import jax
import jax.numpy as jnp
from jax.experimental import pallas as pl


def kernel(vectors, pq_assgin_layer, codebooks):
    raise NotImplementedError("write your pallas kernel here")


# speedup vs baseline: 1.5430x; 1.5430x over previous
import functools

import jax
import jax.numpy as jnp
from jax import lax
from jax.experimental import pallas as pl
from jax.experimental.pallas import tpu as pltpu
from jax.experimental.pallas import tpu_sc as plsc

_B = 512
_NPQ = 1024
_NCB = 128
_NCW = 256
_SUB = 6
_EMB = _NCB * _SUB

_BB = 32


def _pq_select_body(v_ref, pt_ref, n1_ref, n2t_ref, simi_ref, idx_ref):
    v = v_ref[...]
    ip = lax.dot_general(
        v, pt_ref[...], (((1,), (0,)), ((), ())),
        preferred_element_type=jnp.float32)
    simi = -(n1_ref[...] + n2t_ref[...] - 2.0 * ip)
    simi_ref[...] = simi
    m = jnp.max(simi, axis=-1, keepdims=True)
    iota = lax.broadcasted_iota(jnp.int32, simi.shape, 1)
    idx_ref[...] = jnp.min(
        jnp.where(simi == m, iota, _NPQ), axis=-1, keepdims=True)


def _pq_select(vectors, p_t, n1, n2t):
    return pl.pallas_call(
        _pq_select_body,
        out_shape=(
            jax.ShapeDtypeStruct((_B, _NPQ), jnp.float32),
            jax.ShapeDtypeStruct((_B, 1), jnp.int32),
        ),
    )(vectors, p_t, n1, n2t)


def _gather_rows_sc(table, idx):
    rows, d = table.shape
    (b,) = idx.shape
    info = plsc.get_sparse_core_info()
    nw = info.num_cores * info.num_subcores
    b_per_w = b // nw
    mesh = plsc.VectorSubcoreMesh(core_axis_name="c", subcore_axis_name="s")

    @functools.partial(
        pl.kernel,
        mesh=mesh,
        out_type=jax.ShapeDtypeStruct((b, d), jnp.float32),
        scratch_types=[
            pltpu.VMEM((b_per_w,), jnp.int32),
            pltpu.VMEM((b_per_w, d), jnp.float32),
            pltpu.SemaphoreType.DMA,
        ],
    )
    def k(table_hbm, idx_hbm, out_hbm, idx_v, rows_v, sem):
        wid = lax.axis_index("s") * info.num_cores + lax.axis_index("c")
        base = wid * b_per_w
        pltpu.sync_copy(idx_hbm.at[pl.ds(base, b_per_w)], idx_v)
        pltpu.async_copy(table_hbm.at[idx_v], rows_v, sem).wait()
        pltpu.sync_copy(rows_v, out_hbm.at[pl.ds(base, b_per_w)])

    return k(table, idx)


def _code_select_body(vs_ref, cs_ref, cb_ref, ca_ref, qs_ref):
    r = [vs_ref[s] - cs_ref[s] for s in range(_SUB)]
    acc = None
    for s in range(_SUB):
        d = r[s][:, :, None] - cb_ref[s][None, :, :]
        sq = d * d
        acc = sq if acc is None else acc + sq
    proba = -acc
    x = proba / 0.01
    m = jnp.max(x, axis=-1, keepdims=True)
    e = jnp.exp(x - m)
    ca = e / jnp.sum(e, axis=-1, keepdims=True)
    ca_ref[...] = ca
    iota = lax.broadcasted_iota(jnp.int32, ca.shape, 2)
    mca = jnp.max(ca, axis=-1, keepdims=True)
    idx = jnp.min(jnp.where(ca == mca, iota, _NCW), axis=-1)
    oh = (iota == idx[:, :, None]).astype(jnp.float32)
    qn = []
    for s in range(_SUB):
        quant_s = jnp.sum(oh * cb_ref[s][None, :, :], axis=-1)
        qn.append(cs_ref[s] + quant_s)
    sumsq = qn[0] * qn[0]
    for s in range(1, _SUB):
        sumsq = sumsq + qn[s] * qn[s]
    denom = jnp.clip(jnp.sqrt(jnp.sum(sumsq, axis=-1, keepdims=True)),
                     1e-12, None)
    for s in range(_SUB):
        qs_ref[s] = qn[s] / denom


def _code_select(vs, cs, cbt):
    nblk = _B // _BB
    return pl.pallas_call(
        _code_select_body,
        grid=(nblk,),
        in_specs=[
            pl.BlockSpec((_SUB, _BB, _NCB), lambda i: (0, i, 0)),
            pl.BlockSpec((_SUB, _BB, _NCB), lambda i: (0, i, 0)),
            pl.BlockSpec((_SUB, _NCB, _NCW), lambda i: (0, 0, 0)),
        ],
        out_specs=(
            pl.BlockSpec((_BB, _NCB, _NCW), lambda i: (i, 0, 0)),
            pl.BlockSpec((_SUB, _BB, _NCB), lambda i: (0, i, 0)),
        ),
        out_shape=(
            jax.ShapeDtypeStruct((_B, _NCB, _NCW), jnp.float32),
            jax.ShapeDtypeStruct((_SUB, _B, _NCB), jnp.float32),
        ),
    )(vs, cs, cbt)


def kernel(vectors, pq_assgin_layer, codebooks):
    n1 = jnp.sum(vectors ** 2, axis=-1, keepdims=True)
    n2 = jnp.sum(pq_assgin_layer ** 2, axis=-1, keepdims=True)
    simi, idx = _pq_select(vectors, pq_assgin_layer.T, n1, n2.T)
    pq_centers = _gather_rows_sc(pq_assgin_layer, idx.reshape(_B))
    vs = vectors.reshape(_B, _NCB, _SUB).transpose(2, 0, 1)
    cs = pq_centers.reshape(_B, _NCB, _SUB).transpose(2, 0, 1)
    cbt = codebooks.transpose(2, 0, 1)
    ca, qs = _code_select(vs, cs, cbt)
    q = qs.transpose(1, 2, 0).reshape(_B, _EMB)
    return (simi, pq_centers, ca, q)


# speedup vs baseline: 1.8059x; 1.1704x over previous
import functools

import jax
import jax.numpy as jnp
from jax import lax
from jax.experimental import pallas as pl
from jax.experimental.pallas import tpu as pltpu
from jax.experimental.pallas import tpu_sc as plsc

_B = 512
_NPQ = 1024
_NCB = 128
_NCW = 256
_SUB = 6
_EMB = _NCB * _SUB

_BB = 32


def _pq_select_body(v_ref, pt_ref, n1_ref, n2t_ref, simi_ref, idx_ref):
    v = v_ref[...]
    ip = lax.dot_general(
        v, pt_ref[...], (((1,), (0,)), ((), ())),
        preferred_element_type=jnp.float32)
    simi = -(n1_ref[...] + n2t_ref[...] - 2.0 * ip)
    simi_ref[...] = simi
    m = jnp.max(simi, axis=-1, keepdims=True)
    iota = lax.broadcasted_iota(jnp.int32, simi.shape, 1)
    idx_ref[...] = jnp.min(
        jnp.where(simi == m, iota, _NPQ), axis=-1, keepdims=True)


def _pq_select(vectors, p_t, n1, n2t):
    return pl.pallas_call(
        _pq_select_body,
        out_shape=(
            jax.ShapeDtypeStruct((_B, _NPQ), jnp.float32),
            jax.ShapeDtypeStruct((_B, 1), jnp.int32),
        ),
    )(vectors, p_t, n1, n2t)


def _gather_rows_sc(table, idx):
    rows, d = table.shape
    (b,) = idx.shape
    info = plsc.get_sparse_core_info()
    nw = info.num_cores * info.num_subcores
    b_per_w = b // nw
    mesh = plsc.VectorSubcoreMesh(core_axis_name="c", subcore_axis_name="s")

    @functools.partial(
        pl.kernel,
        mesh=mesh,
        out_type=jax.ShapeDtypeStruct((b, d), jnp.float32),
        scratch_types=[
            pltpu.VMEM((b_per_w,), jnp.int32),
            pltpu.VMEM((b_per_w, d), jnp.float32),
            pltpu.SemaphoreType.DMA,
        ],
    )
    def k(table_hbm, idx_hbm, out_hbm, idx_v, rows_v, sem):
        wid = lax.axis_index("s") * info.num_cores + lax.axis_index("c")
        base = wid * b_per_w
        pltpu.sync_copy(idx_hbm.at[pl.ds(base, b_per_w)], idx_v)
        pltpu.async_copy(table_hbm.at[idx_v], rows_v, sem).wait()
        pltpu.sync_copy(rows_v, out_hbm.at[pl.ds(base, b_per_w)])

    return k(table, idx)


def _code_select_body(vs_ref, cs_ref, cb_ref, ca_ref, qs_ref):
    r2 = [(vs_ref[s] - cs_ref[s]) * 200.0 for s in range(_SUB)]
    cn2 = None
    for s in range(_SUB):
        c = cb_ref[s]
        cn2 = c * c if cn2 is None else cn2 + c * c
    x = cn2 * -100.0
    for s in range(_SUB):
        x = x + r2[s][:, :, None] * cb_ref[s][None, :, :]
    m = jnp.max(x, axis=-1, keepdims=True)
    e = jnp.exp(x - m)
    rcp = 1.0 / jnp.sum(e, axis=-1, keepdims=True)
    ca_ref[...] = e * rcp
    iota = lax.broadcasted_iota(jnp.int32, x.shape, 2)
    idx = jnp.min(jnp.where(x == m, iota, _NCW), axis=-1)
    idx_t = idx.T
    hi = idx_t >= 128
    idx_lo = jnp.where(hi, 0, idx_t)
    idx_hi = jnp.where(hi, idx_t - 128, 0)
    qn_t = []
    for s in range(_SUB):
        g_lo = jnp.take_along_axis(cb_ref[s][:, :128], idx_lo, axis=1)
        g_hi = jnp.take_along_axis(cb_ref[s][:, 128:], idx_hi, axis=1)
        quant_t = jnp.where(hi, g_hi, g_lo)
        qn_t.append(cs_ref[s].T + quant_t)
    sumsq = qn_t[0] * qn_t[0]
    for s in range(1, _SUB):
        sumsq = sumsq + qn_t[s] * qn_t[s]
    denom = jnp.clip(jnp.sqrt(jnp.sum(sumsq, axis=0, keepdims=True)),
                     1e-12, None)
    for s in range(_SUB):
        qs_ref[0, s] = qn_t[s] / denom


def _code_select(vs, cs, cbt):
    nblk = _B // _BB
    return pl.pallas_call(
        _code_select_body,
        grid=(nblk,),
        in_specs=[
            pl.BlockSpec((_SUB, _BB, _NCB), lambda i: (0, i, 0)),
            pl.BlockSpec((_SUB, _BB, _NCB), lambda i: (0, i, 0)),
            pl.BlockSpec((_SUB, _NCB, _NCW), lambda i: (0, 0, 0)),
        ],
        out_specs=(
            pl.BlockSpec((_BB, _NCB, _NCW), lambda i: (i, 0, 0)),
            pl.BlockSpec((1, _SUB, _NCB, _BB), lambda i: (i, 0, 0, 0)),
        ),
        out_shape=(
            jax.ShapeDtypeStruct((_B, _NCB, _NCW), jnp.float32),
            jax.ShapeDtypeStruct((_B // _BB, _SUB, _NCB, _BB), jnp.float32),
        ),
    )(vs, cs, cbt)


def kernel(vectors, pq_assgin_layer, codebooks):
    n1 = jnp.sum(vectors ** 2, axis=-1, keepdims=True)
    n2 = jnp.sum(pq_assgin_layer ** 2, axis=-1, keepdims=True)
    simi, idx = _pq_select(vectors, pq_assgin_layer.T, n1, n2.T)
    pq_centers = _gather_rows_sc(pq_assgin_layer, idx.reshape(_B))
    vs = vectors.reshape(_B, _NCB, _SUB).transpose(2, 0, 1)
    cs = pq_centers.reshape(_B, _NCB, _SUB).transpose(2, 0, 1)
    cbt = codebooks.transpose(2, 0, 1)
    ca, qs = _code_select(vs, cs, cbt)
    q = qs.transpose(0, 3, 2, 1).reshape(_B, _EMB)
    return (simi, pq_centers, ca, q)


# speedup vs baseline: 2.0202x; 1.1186x over previous
import functools

import jax
import jax.numpy as jnp
from jax import lax
from jax.experimental import pallas as pl
from jax.experimental.pallas import tpu as pltpu
from jax.experimental.pallas import tpu_sc as plsc

_B = 512
_NPQ = 1024
_NCB = 128
_NCW = 256
_SUB = 6
_EMB = _NCB * _SUB

_BB = 64


def _pq_select_body(v_ref, p_ref, n1_ref, n2t_ref, simi_ref, idx_ref):
    v = v_ref[...]
    ip = lax.dot_general(
        v, p_ref[...], (((1,), (1,)), ((), ())),
        preferred_element_type=jnp.float32)
    simi = -(n1_ref[...] + n2t_ref[...] - 2.0 * ip)
    simi_ref[...] = simi
    m = jnp.max(simi, axis=-1, keepdims=True)
    iota = lax.broadcasted_iota(jnp.int32, simi.shape, 1)
    idx_ref[...] = jnp.min(
        jnp.where(simi == m, iota, _NPQ), axis=-1, keepdims=True)


def _pq_select(vectors, p_t, n1, n2t):
    return pl.pallas_call(
        _pq_select_body,
        out_shape=(
            jax.ShapeDtypeStruct((_B, _NPQ), jnp.float32),
            jax.ShapeDtypeStruct((_B, 1), jnp.int32),
        ),
    )(vectors, p_t, n1, n2t)


def _gather_rows_sc(table, idx):
    rows, d = table.shape
    (b,) = idx.shape
    info = plsc.get_sparse_core_info()
    nw = info.num_cores * info.num_subcores
    b_per_w = b // nw
    mesh = plsc.VectorSubcoreMesh(core_axis_name="c", subcore_axis_name="s")

    @functools.partial(
        pl.kernel,
        mesh=mesh,
        out_type=jax.ShapeDtypeStruct((b, d), jnp.float32),
        scratch_types=[
            pltpu.VMEM((b_per_w,), jnp.int32),
            pltpu.VMEM((b_per_w, d), jnp.float32),
            pltpu.SemaphoreType.DMA,
        ],
    )
    def k(table_hbm, idx_hbm, out_hbm, idx_v, rows_v, sem):
        wid = lax.axis_index("s") * info.num_cores + lax.axis_index("c")
        base = wid * b_per_w
        pltpu.sync_copy(idx_hbm.at[pl.ds(base, b_per_w)], idx_v)
        pltpu.async_copy(table_hbm.at[idx_v], rows_v, sem).wait()
        pltpu.sync_copy(rows_v, out_hbm.at[pl.ds(base, b_per_w)])

    return k(table, idx)


def _code_select_body(vs_ref, cs_ref, cb_ref, ca_ref, qs_ref):
    r2 = [(vs_ref[s] - cs_ref[s]) * 200.0 for s in range(_SUB)]
    cn2 = None
    for s in range(_SUB):
        c = cb_ref[s]
        cn2 = c * c if cn2 is None else cn2 + c * c
    x = cn2 * -100.0
    for s in range(_SUB):
        x = x + r2[s][:, :, None] * cb_ref[s][None, :, :]
    m = jnp.max(x, axis=-1, keepdims=True)
    e = jnp.exp(x - m)
    rcp = 1.0 / jnp.sum(e, axis=-1, keepdims=True)
    ca_ref[...] = e * rcp
    iota = lax.broadcasted_iota(jnp.int32, x.shape, 2)
    idx = jnp.min(jnp.where(x == m, iota, _NCW), axis=-1)
    idx_t = idx.T
    hi = idx_t >= 128
    idx_lo = jnp.where(hi, 0, idx_t)
    idx_hi = jnp.where(hi, idx_t - 128, 0)
    qn_t = []
    for s in range(_SUB):
        g_lo = jnp.take_along_axis(cb_ref[s][:, :128], idx_lo, axis=1)
        g_hi = jnp.take_along_axis(cb_ref[s][:, 128:], idx_hi, axis=1)
        quant_t = jnp.where(hi, g_hi, g_lo)
        qn_t.append(cs_ref[s].T + quant_t)
    sumsq = qn_t[0] * qn_t[0]
    for s in range(1, _SUB):
        sumsq = sumsq + qn_t[s] * qn_t[s]
    denom = jnp.clip(jnp.sqrt(jnp.sum(sumsq, axis=0, keepdims=True)),
                     1e-12, None)
    for s in range(_SUB):
        qs_ref[0, s] = qn_t[s] / denom


def _code_select(vs, cs, cbt):
    nblk = _B // _BB
    return pl.pallas_call(
        _code_select_body,
        grid=(nblk,),
        in_specs=[
            pl.BlockSpec((_SUB, _BB, _NCB), lambda i: (0, i, 0)),
            pl.BlockSpec((_SUB, _BB, _NCB), lambda i: (0, i, 0)),
            pl.BlockSpec((_SUB, _NCB, _NCW), lambda i: (0, 0, 0)),
        ],
        out_specs=(
            pl.BlockSpec((_BB, _NCB, _NCW), lambda i: (i, 0, 0)),
            pl.BlockSpec((1, _SUB, _NCB, _BB), lambda i: (i, 0, 0, 0)),
        ),
        out_shape=(
            jax.ShapeDtypeStruct((_B, _NCB, _NCW), jnp.float32),
            jax.ShapeDtypeStruct((_B // _BB, _SUB, _NCB, _BB), jnp.float32),
        ),
    )(vs, cs, cbt)


def kernel(vectors, pq_assgin_layer, codebooks):
    n1 = jnp.sum(vectors ** 2, axis=-1, keepdims=True)
    n2 = jnp.sum(pq_assgin_layer ** 2, axis=-1, keepdims=True)
    simi, idx = _pq_select(vectors, pq_assgin_layer, n1, n2.T)
    pq_centers = _gather_rows_sc(pq_assgin_layer, idx.reshape(_B))
    vs = vectors.reshape(_B, _NCB, _SUB).transpose(2, 0, 1)
    cs = pq_centers.reshape(_B, _NCB, _SUB).transpose(2, 0, 1)
    cbt = codebooks.transpose(2, 0, 1)
    ca, qs = _code_select(vs, cs, cbt)
    q = qs.transpose(0, 3, 2, 1).reshape(_B, _EMB)
    return (simi, pq_centers, ca, q)
